# Initial kernel scaffold; baseline (speedup 1.0000x reference)
import jax
import jax.numpy as jnp
from jax.experimental import pallas as pl


def kernel(x, edge_index, batch, W1, b1, gn_w1, gn_b1, gn_ms1, W2, b2, gn_w2, gn_b2, gn_ms2, W3, b3, gn_w3, gn_b3, gn_ms3, gW1, gb1, gW2, gb2):
    raise NotImplementedError("write your pallas kernel here")


# speedup vs baseline: 16.2286x; 16.2286x over previous
import functools

import jax
import jax.numpy as jnp
from jax import lax
from jax.experimental import pallas as pl
from jax.experimental.pallas import tpu as pltpu
from jax.experimental.pallas import tpu_sc as plsc

N = 10000
E = 320000
D = 128
B = 64
GATE_H = 128

NC = 2
NS = 16
NW = NC * NS
EPW = E // NW
CHUNK = 80
NCHUNK = EPW // CHUNK
NPAD = 10240
RPT = NPAD // NS
DW = 128


def _deg_body(dst_hbm, zero8_hbm, ones8_hbm, out_hbm, didx, ones_v, hist):
    c = lax.axis_index("c")
    s = lax.axis_index("s")
    wid = c * NS + s
    r0 = s * RPT
    pltpu.sync_copy(zero8_hbm, hist.at[pl.ds(r0, RPT)])
    pltpu.sync_copy(ones8_hbm, ones_v)
    pltpu.sync_copy(dst_hbm.at[wid], didx)
    plsc.subcore_barrier()

    def body(j, carry):
        pltpu.sync_copy(ones_v, hist.at[didx.at[j]], add=True)
        return carry

    lax.fori_loop(0, NCHUNK, body, 0)
    plsc.subcore_barrier()
    pltpu.sync_copy(hist.at[pl.ds(r0, RPT)], out_hbm.at[c, pl.ds(r0, RPT)])


def _scatter_body(y_hbm, src_hbm, dst_hbm, zero_hbm, out_hbm, sidx, didx, rows,
                  acc, sem):
    c = lax.axis_index("c")
    s = lax.axis_index("s")
    wid = c * NS + s
    r0 = s * RPT
    pltpu.sync_copy(zero_hbm, acc.at[pl.ds(r0, RPT)])
    pltpu.sync_copy(src_hbm.at[wid], sidx)
    pltpu.sync_copy(dst_hbm.at[wid], didx)
    plsc.subcore_barrier()

    def body(j, carry):
        pltpu.async_copy(y_hbm.at[sidx.at[j]], rows, sem).wait()
        pltpu.sync_copy(rows, acc.at[didx.at[j]], add=True)
        return carry

    lax.fori_loop(0, NCHUNK, body, 0)
    plsc.subcore_barrier()
    pltpu.sync_copy(acc.at[pl.ds(r0, RPT)], out_hbm.at[c, pl.ds(r0, RPT)])


@functools.lru_cache(maxsize=None)
def _sc_kernels():
    mesh = plsc.VectorSubcoreMesh(core_axis_name="c", subcore_axis_name="s",
                                  num_cores=NC, num_subcores=NS)
    deg = pl.kernel(
        _deg_body,
        out_type=jax.ShapeDtypeStruct((NC, NPAD, DW), jnp.float32),
        mesh=mesh,
        scratch_types=[
            pltpu.VMEM((NCHUNK, CHUNK), jnp.int32),
            pltpu.VMEM((CHUNK, DW), jnp.float32),
            pltpu.VMEM_SHARED((NPAD, DW), jnp.float32),
        ],
    )
    scatter = pl.kernel(
        _scatter_body,
        out_type=jax.ShapeDtypeStruct((NC, NPAD, D), jnp.float32),
        mesh=mesh,
        scratch_types=[
            pltpu.VMEM((NCHUNK, CHUNK), jnp.int32),
            pltpu.VMEM((NCHUNK, CHUNK), jnp.int32),
            pltpu.VMEM((CHUNK, D), jnp.float32),
            pltpu.VMEM_SHARED((NPAD, D), jnp.float32),
            pltpu.SemaphoreType.DMA,
        ],
    )
    return deg, scatter


def _f32dot(a, bm, dn=None):
    if dn is None:
        return jnp.dot(a, bm, preferred_element_type=jnp.float32)
    return lax.dot_general(a, bm, (dn, ((), ())),
                           preferred_element_type=jnp.float32)


def _prep_body(x_ref, w1_ref, hist_ref, y_ref, dinv_ref):
    deg = hist_ref[0, :N, 0:1] + hist_ref[1, :N, 0:1] + 1.0
    dinv = lax.rsqrt(deg)
    h = _f32dot(x_ref[...], w1_ref[...])
    y_ref[...] = h * dinv
    dinv_ref[...] = dinv


def _graph_norm_relu(conv, batch2, gw, gb, gms):
    mT = batch2 == lax.broadcasted_iota(jnp.int32, (N, B), 1)
    mTf = mT.astype(jnp.float32)
    onesN = jnp.ones((N, 1), jnp.float32)
    cnt = jnp.maximum(_f32dot(mTf, onesN, ((0,), (0,))), 1.0)
    sx = _f32dot(mTf, conv, ((0,), (0,)))
    mean_ms = (sx / cnt) * gms
    xc = conv - _f32dot(mTf, mean_ms)
    sxx = _f32dot(mTf, xc * xc, ((0,), (0,)))
    istd = lax.rsqrt(sxx / cnt + 1e-5)
    xn = xc * _f32dot(mTf, istd)
    return jnp.maximum(xn * gw + gb, 0.0)


def _layer_body(z_ref, y_ref, dinv_ref, b_ref, gw_ref, gb_ref, gms_ref,
                batch_ref, wn_ref, out_ref):
    dinv = dinv_ref[...]
    conv = dinv * (z_ref[0, :N, :] + z_ref[1, :N, :] + y_ref[...]) + b_ref[...]
    act = _graph_norm_relu(conv, batch_ref[...], gw_ref[...], gb_ref[...],
                           gms_ref[...])
    out_ref[...] = _f32dot(act, wn_ref[...]) * dinv


def _final_body(z_ref, y_ref, dinv_ref, b_ref, gw_ref, gb_ref, gms_ref,
                batch_ref, gw1_ref, gb1_ref, gw2_ref, gb2_ref, out_ref):
    dinv = dinv_ref[...]
    conv = dinv * (z_ref[0, :N, :] + z_ref[1, :N, :] + y_ref[...]) + b_ref[...]
    x3 = _graph_norm_relu(conv, batch_ref[...], gw_ref[...], gb_ref[...],
                          gms_ref[...])
    g1 = jnp.maximum(_f32dot(x3, gw1_ref[...]) + gb1_ref[...], 0.0)
    g = _f32dot(g1, gw2_ref[...]) + gb2_ref[...]
    mT = batch_ref[...] == lax.broadcasted_iota(jnp.int32, (N, B), 1)
    neg = jnp.float32(-jnp.inf)
    gm = jnp.max(jnp.where(mT, g, neg), axis=0, keepdims=True)
    ge = jnp.where(mT, jnp.exp(g - gm), 0.0)
    onesN = jnp.ones((N, 1), jnp.float32)
    gsum = _f32dot(ge, onesN, ((0,), (0,)))
    wsum = _f32dot(ge, x3, ((0,), (0,)))
    out_ref[...] = wsum / (gsum + 1e-16)


def _tc(body, out_shape, *args):
    return pl.pallas_call(body, out_shape=out_shape)(*args)


def kernel(x, edge_index, batch, W1, b1, gn_w1, gn_b1, gn_ms1, W2, b2, gn_w2,
           gn_b2, gn_ms2, W3, b3, gn_w3, gn_b3, gn_ms3, gW1, gb1, gW2, gb2):
    f32 = jnp.float32
    src3 = edge_index[0].reshape(NW, NCHUNK, CHUNK)
    dst3 = edge_index[1].reshape(NW, NCHUNK, CHUNK)
    batch2 = batch.reshape(N, 1)
    zeroD = jnp.zeros((RPT, D), f32)
    zero8 = jnp.zeros((RPT, DW), f32)
    ones8 = jnp.ones((CHUNK, DW), f32)

    sc_deg, sc_scatter = _sc_kernels()
    hist = sc_deg(dst3, zero8, ones8)
    y1, dinv = _tc(
        _prep_body,
        (jax.ShapeDtypeStruct((N, D), f32), jax.ShapeDtypeStruct((N, 1), f32)),
        x, W1, hist)

    r = lambda v: v.reshape(1, -1)
    z = sc_scatter(y1, src3, dst3, zeroD)
    y2 = _tc(_layer_body, jax.ShapeDtypeStruct((N, D), f32),
             z, y1, dinv, r(b1), r(gn_w1), r(gn_b1), r(gn_ms1), batch2, W2)
    z = sc_scatter(y2, src3, dst3, zeroD)
    y3 = _tc(_layer_body, jax.ShapeDtypeStruct((N, D), f32),
             z, y2, dinv, r(b2), r(gn_w2), r(gn_b2), r(gn_ms2), batch2, W3)
    z = sc_scatter(y3, src3, dst3, zeroD)
    out = _tc(_final_body, jax.ShapeDtypeStruct((B, D), f32),
              z, y3, dinv, r(b3), r(gn_w3), r(gn_b3), r(gn_ms3), batch2,
              gW1, r(gb1), gW2, r(gb2))
    return out


# speedup vs baseline: 26.0406x; 1.6046x over previous
import functools

import jax
import jax.numpy as jnp
from jax import lax
from jax.experimental import pallas as pl
from jax.experimental.pallas import tpu as pltpu
from jax.experimental.pallas import tpu_sc as plsc

N = 10000
E = 320000
D = 128
B = 64
GATE_H = 128

NC = 2
NS = 16
NW = NC * NS
EPW = E // NW
FCH = 128
NF = EPW // FCH
NPAIR = NF // 2
TCH = EPW - NF * FCH
NPAD = 10240
RPT = NPAD // NS
HR = 128
HC = 128


def _deg_body(dst_hbm, zeroH_hbm, iidx_hbm, out_hbm, didx, hist_v, iidx_v,
              hist_s):
    c = lax.axis_index("c")
    s = lax.axis_index("s")
    wid = c * NS + s
    pltpu.sync_copy(zeroH_hbm, hist_v)
    pltpu.sync_copy(iidx_hbm, iidx_v)
    pltpu.sync_copy(dst_hbm.at[pl.ds(wid * EPW, EPW)], didx)
    pltpu.sync_copy(zeroH_hbm.at[pl.ds(0, HR // NS)],
                    hist_s.at[pl.ds(s * (HR // NS), HR // NS)])
    ones16 = jnp.full((16,), 1.0, jnp.float32)

    def body(i, carry):
        v = didx[pl.ds(i * 16, 16)]
        plsc.addupdate_scatter(hist_v, [v // HC, v % HC], ones16)
        return carry

    lax.fori_loop(0, EPW // 16, body, 0)
    plsc.subcore_barrier()
    pltpu.sync_copy(hist_v, hist_s.at[iidx_v.at[0]], add=True)
    plsc.subcore_barrier()
    pltpu.sync_copy(hist_s.at[pl.ds(s * (HR // NS), HR // NS)],
                    out_hbm.at[c, pl.ds(s * (HR // NS), HR // NS)])


def _scatter_body(y_hbm, src_hbm, dst_hbm, zero_hbm, out_hbm,
                  sidx0, didx0, sidx1, didx1, rows0, rows1,
                  sidxT, didxT, rowsT, acc,
                  isem0, isem1, gsem0, gsem1):
    c = lax.axis_index("c")
    s = lax.axis_index("s")
    wid = c * NS + s
    r0 = s * RPT
    ebase = wid * EPW

    def load_idx(j, sbuf, dbuf, sem):
        pltpu.async_copy(src_hbm.at[pl.ds(ebase + j * FCH, FCH)], sbuf, sem)
        pltpu.async_copy(dst_hbm.at[pl.ds(ebase + j * FCH, FCH)], dbuf, sem)

    def wait_idx(sbuf, dbuf, sem):
        pltpu.make_async_copy(src_hbm.at[pl.ds(ebase, FCH)], sbuf, sem).wait()
        pltpu.make_async_copy(src_hbm.at[pl.ds(ebase, FCH)], dbuf, sem).wait()

    pltpu.sync_copy(zero_hbm, acc.at[pl.ds(r0, RPT)])
    load_idx(0, sidx0, didx0, isem0)
    load_idx(1, sidx1, didx1, isem1)
    wait_idx(sidx0, didx0, isem0)
    pltpu.async_copy(y_hbm.at[sidx0], rows0, gsem0)
    plsc.subcore_barrier()

    def body(t, carry):
        j0 = 2 * t
        wait_idx(sidx1, didx1, isem1)
        pltpu.async_copy(y_hbm.at[sidx1], rows1, gsem1)
        pltpu.make_async_copy(y_hbm.at[sidx0], rows0, gsem0).wait()
        pltpu.sync_copy(rows0, acc.at[didx0], add=True)

        @pl.when(j0 + 2 < NF)
        def _():
            load_idx(j0 + 2, sidx0, didx0, isem0)
            wait_idx(sidx0, didx0, isem0)
            pltpu.async_copy(y_hbm.at[sidx0], rows0, gsem0)

        pltpu.make_async_copy(y_hbm.at[sidx1], rows1, gsem1).wait()
        pltpu.sync_copy(rows1, acc.at[didx1], add=True)

        @pl.when(j0 + 3 < NF)
        def _():
            load_idx(j0 + 3, sidx1, didx1, isem1)

        return carry

    lax.fori_loop(0, NPAIR, body, 0)

    pltpu.sync_copy(src_hbm.at[pl.ds(ebase + NF * FCH, TCH)], sidxT)
    pltpu.sync_copy(dst_hbm.at[pl.ds(ebase + NF * FCH, TCH)], didxT)
    pltpu.async_copy(y_hbm.at[sidxT], rowsT, gsem0).wait()
    pltpu.sync_copy(rowsT, acc.at[didxT], add=True)

    plsc.subcore_barrier()
    pltpu.sync_copy(acc.at[pl.ds(r0, RPT)], out_hbm.at[c, pl.ds(r0, RPT)])


@functools.lru_cache(maxsize=None)
def _sc_kernels():
    mesh = plsc.VectorSubcoreMesh(core_axis_name="c", subcore_axis_name="s",
                                  num_cores=NC, num_subcores=NS)
    deg = pl.kernel(
        _deg_body,
        out_type=jax.ShapeDtypeStruct((NC, HR, HC), jnp.float32),
        mesh=mesh,
        compiler_params=pltpu.CompilerParams(needs_layout_passes=False),
        scratch_types=[
            pltpu.VMEM((EPW,), jnp.int32),
            pltpu.VMEM((HR, HC), jnp.float32),
            pltpu.VMEM((1, HR), jnp.int32),
            pltpu.VMEM_SHARED((HR, HC), jnp.float32),
        ],
    )
    scatter = pl.kernel(
        _scatter_body,
        out_type=jax.ShapeDtypeStruct((NC, NPAD, D), jnp.float32),
        mesh=mesh,
        scratch_types=[
            pltpu.VMEM((FCH,), jnp.int32),
            pltpu.VMEM((FCH,), jnp.int32),
            pltpu.VMEM((FCH,), jnp.int32),
            pltpu.VMEM((FCH,), jnp.int32),
            pltpu.VMEM((FCH, D), jnp.float32),
            pltpu.VMEM((FCH, D), jnp.float32),
            pltpu.VMEM((TCH,), jnp.int32),
            pltpu.VMEM((TCH,), jnp.int32),
            pltpu.VMEM((TCH, D), jnp.float32),
            pltpu.VMEM_SHARED((NPAD, D), jnp.float32),
            pltpu.SemaphoreType.DMA,
            pltpu.SemaphoreType.DMA,
            pltpu.SemaphoreType.DMA,
            pltpu.SemaphoreType.DMA,
        ],
    )
    return deg, scatter


def _f32dot(a, bm, dn=None):
    if dn is None:
        return jnp.dot(a, bm, preferred_element_type=jnp.float32)
    return lax.dot_general(a, bm, (dn, ((), ())),
                           preferred_element_type=jnp.float32)


def _prep_body(x_ref, w1_ref, hist_ref, y_ref, dinv_ref):
    deg = hist_ref[0] + hist_ref[1] + 1.0
    dinv = lax.rsqrt(deg)
    h = _f32dot(x_ref[...], w1_ref[...])
    y_ref[...] = h * dinv
    dinv_ref[...] = dinv


def _graph_norm_relu(conv, batch2, gw, gb, gms):
    mT = batch2 == lax.broadcasted_iota(jnp.int32, (N, B), 1)
    mTf = mT.astype(jnp.float32)
    onesN = jnp.ones((N, 1), jnp.float32)
    cnt = jnp.maximum(_f32dot(mTf, onesN, ((0,), (0,))), 1.0)
    sx = _f32dot(mTf, conv, ((0,), (0,)))
    mean_ms = (sx / cnt) * gms
    xc = conv - _f32dot(mTf, mean_ms)
    sxx = _f32dot(mTf, xc * xc, ((0,), (0,)))
    istd = lax.rsqrt(sxx / cnt + 1e-5)
    xn = xc * _f32dot(mTf, istd)
    return jnp.maximum(xn * gw + gb, 0.0)


def _layer_body(z_ref, y_ref, dinv_ref, b_ref, gw_ref, gb_ref, gms_ref,
                batch_ref, wn_ref, out_ref):
    dinv = dinv_ref[...]
    conv = dinv * (z_ref[0, :N, :] + z_ref[1, :N, :] + y_ref[...]) + b_ref[...]
    act = _graph_norm_relu(conv, batch_ref[...], gw_ref[...], gb_ref[...],
                           gms_ref[...])
    out_ref[...] = _f32dot(act, wn_ref[...]) * dinv


def _final_body(z_ref, y_ref, dinv_ref, b_ref, gw_ref, gb_ref, gms_ref,
                batch_ref, gw1_ref, gb1_ref, gw2_ref, gb2_ref, out_ref):
    dinv = dinv_ref[...]
    conv = dinv * (z_ref[0, :N, :] + z_ref[1, :N, :] + y_ref[...]) + b_ref[...]
    x3 = _graph_norm_relu(conv, batch_ref[...], gw_ref[...], gb_ref[...],
                          gms_ref[...])
    g1 = jnp.maximum(_f32dot(x3, gw1_ref[...]) + gb1_ref[...], 0.0)
    g = _f32dot(g1, gw2_ref[...]) + gb2_ref[...]
    mT = batch_ref[...] == lax.broadcasted_iota(jnp.int32, (N, B), 1)
    neg = jnp.float32(-jnp.inf)
    gm = jnp.max(jnp.where(mT, g, neg), axis=0, keepdims=True)
    ge = jnp.where(mT, jnp.exp(g - gm), 0.0)
    onesN = jnp.ones((N, 1), jnp.float32)
    gsum = _f32dot(ge, onesN, ((0,), (0,)))
    wsum = _f32dot(ge, x3, ((0,), (0,)))
    out_ref[...] = wsum / (gsum + 1e-16)


def _tc(body, out_shape, *args):
    return pl.pallas_call(body, out_shape=out_shape)(*args)


def kernel(x, edge_index, batch, W1, b1, gn_w1, gn_b1, gn_ms1, W2, b2, gn_w2,
           gn_b2, gn_ms2, W3, b3, gn_w3, gn_b3, gn_ms3, gW1, gb1, gW2, gb2):
    f32 = jnp.float32
    batch2 = batch.reshape(N, 1)
    zeroD = jnp.zeros((RPT, D), f32)
    zeroH = jnp.zeros((HR, HC), f32)
    iidxH = jnp.arange(HR, dtype=jnp.int32).reshape(1, HR)

    sc_deg, sc_scatter = _sc_kernels()
    hist = sc_deg(edge_index[1], zeroH, iidxH)
    hist2 = hist.reshape(NC, HR * HC)[:, :N].reshape(NC, N, 1)
    y1, dinv = _tc(
        _prep_body,
        (jax.ShapeDtypeStruct((N, D), f32), jax.ShapeDtypeStruct((N, 1), f32)),
        x, W1, hist2)

    r = lambda v: v.reshape(1, -1)
    z = sc_scatter(y1, edge_index[0], edge_index[1], zeroD)
    y2 = _tc(_layer_body, jax.ShapeDtypeStruct((N, D), f32),
             z, y1, dinv, r(b1), r(gn_w1), r(gn_b1), r(gn_ms1), batch2, W2)
    z = sc_scatter(y2, edge_index[0], edge_index[1], zeroD)
    y3 = _tc(_layer_body, jax.ShapeDtypeStruct((N, D), f32),
             z, y2, dinv, r(b2), r(gn_w2), r(gn_b2), r(gn_ms2), batch2, W3)
    z = sc_scatter(y3, edge_index[0], edge_index[1], zeroD)
    out = _tc(_final_body, jax.ShapeDtypeStruct((B, D), f32),
              z, y3, dinv, r(b3), r(gn_w3), r(gn_b3), r(gn_ms3), batch2,
              gW1, r(gb1), gW2, r(gb2))
    return out
